# baseline (speedup 1.0000x reference)
import functools

import jax
import jax.numpy as jnp
from jax import lax
from jax.experimental import pallas as pl
from jax.experimental.pallas import tpu as pltpu
from jax.experimental.pallas import tpu_sc as plsc

_TOP_K = 8
_N_GROUPS = 8
_GROUP_SIZE = 8
_LANES = 16
_SC_SUB = 256

_SORT8 = (
    (0, 1), (2, 3), (4, 5), (6, 7),
    (0, 2), (1, 3), (4, 6), (5, 7),
    (1, 2), (5, 6), (0, 4), (3, 7),
    (1, 5), (2, 6),
    (1, 4), (3, 6),
    (2, 4), (3, 5),
    (3, 4),
)
_BMERGE8 = (
    (0, 4), (1, 5), (2, 6), (3, 7),
    (0, 2), (1, 3), (4, 6), (5, 7),
    (0, 1), (2, 3), (4, 5), (6, 7),
)


def _gate_softmax_body(x_ref, w_ref, p_ref, *, sc_tok):
    s = lax.dot_general(
        w_ref[...], x_ref[...],
        (((1,), (1,)), ((), ())),
        preferred_element_type=jnp.float32,
    )
    m = jnp.max(s, axis=0, keepdims=True)
    e = jnp.exp(s - m)
    p = e / jnp.sum(e, axis=0, keepdims=True)
    block = p.shape[1]
    p_ref[...] = jnp.concatenate(
        [p[:, j * sc_tok:(j + 1) * sc_tok].reshape(-1, 128)
         for j in range(block // sc_tok)], axis=0)


def _gate_softmax(x, w_gate, tok_base, n_tok, block=1024):
    n, d = x.shape
    n_e = w_gate.shape[0]
    rows_per_block = n_e * block // 128
    base_blocks = tok_base // block
    return pl.pallas_call(
        functools.partial(_gate_softmax_body, sc_tok=_SC_SUB),
        grid=(n_tok // block,),
        in_specs=[
            pl.BlockSpec((block, d), lambda i: (i + base_blocks, 0)),
            pl.BlockSpec((n_e, d), lambda i: (0, 0)),
        ],
        out_specs=pl.BlockSpec((rows_per_block, 128), lambda i: (i, 0)),
        out_shape=jax.ShapeDtypeStruct((n_e * n_tok // 128, 128),
                                       jnp.float32),
    )(x, w_gate)


def _lex_gt(av, ai, bv, bi):
    return (av > bv) | ((av == bv) & (ai < bi))


def _ce(vals, idxs, a, b, lex=True):
    if lex:
        c = _lex_gt(vals[a], idxs[a], vals[b], idxs[b])
    else:
        c = vals[a] > vals[b]
    va = jnp.where(c, vals[a], vals[b])
    vb = jnp.where(c, vals[b], vals[a])
    ia = jnp.where(c, idxs[a], idxs[b])
    ib = jnp.where(c, idxs[b], idxs[a])
    vals[a], vals[b], idxs[a], idxs[b] = va, vb, ia, ib


def _route_sc(p3, n, n_e):
    info = plsc.get_sparse_core_info()
    nw = info.num_cores * info.num_subcores
    tok_w = n // nw
    n_chunks = tok_w // _LANES
    mesh = plsc.VectorSubcoreMesh(core_axis_name="c", subcore_axis_name="s")
    p_flat = p3.reshape(n * n_e)

    @functools.partial(
        pl.kernel,
        mesh=mesh,
        compiler_params=pltpu.CompilerParams(needs_layout_passes=False),
        out_type=[
            jax.ShapeDtypeStruct((n * _TOP_K,), jnp.float32),
            jax.ShapeDtypeStruct((n * _TOP_K,), jnp.int32),
            jax.ShapeDtypeStruct((nw, n_e), jnp.float32),
        ],
        scratch_types=[
            pltpu.VMEM((n_e * tok_w,), jnp.float32),
            pltpu.VMEM((tok_w * _TOP_K,), jnp.float32),
            pltpu.VMEM((tok_w * _TOP_K,), jnp.int32),
            pltpu.VMEM((n_e,), jnp.float32),
            pltpu.SemaphoreType.DMA,
            pltpu.SemaphoreType.DMA,
            pltpu.SemaphoreType.DMA,
            pltpu.SemaphoreType.DMA,
            pltpu.SemaphoreType.DMA,
        ],
    )
    def body(pt_hbm, w_hbm, idx_hbm, cnt_hbm, p_v, wout_v, iout_v, cnt_v,
             s0, s1, s2, s3, so):
        n_sub = tok_w // _SC_SUB
        sub_sz = n_e * _SC_SUB
        in_sems = [s0, s1, s2, s3]
        cid = lax.axis_index("c")
        sid = lax.axis_index("s")
        wid = sid * info.num_cores + cid
        base = wid * tok_w
        in_copies = []
        for q in range(n_sub):
            in_copies.append(pltpu.async_copy(
                pt_hbm.at[pl.ds((wid * n_sub + q) * sub_sz, sub_sz)],
                p_v.at[pl.ds(q * sub_sz, sub_sz)],
                in_sems[q]))
        zeros = jnp.zeros((_LANES,), jnp.float32)
        for j in range(n_e // _LANES):
            cnt_v[pl.ds(j * _LANES, _LANES)] = zeros
        lanes = lax.iota(jnp.int32, _LANES)
        ones = jnp.ones((_LANES,), jnp.float32)
        out_copies = []

        def make_chunk(q):
            sub_base = q * sub_sz

            def chunk(c, carry):
                col = c * _LANES
                rows = col + lanes
                g = []
                for j in range(_N_GROUPS):
                    m = p_v[pl.ds(
                        sub_base + j * _GROUP_SIZE * _SC_SUB + col, _LANES)]
                    for o in range(1, _GROUP_SIZE):
                        m = jnp.maximum(
                            m,
                            p_v[pl.ds(
                                sub_base + (j * _GROUP_SIZE + o) * _SC_SUB
                                + col, _LANES)])
                    g.append(m)
                bv, bi = g[0], jnp.zeros((_LANES,), jnp.int32)
                for j in range(1, _N_GROUPS):
                    c1 = g[j] > bv
                    bv = jnp.where(c1, g[j], bv)
                    bi = jnp.where(c1, j, bi)
                sv = jnp.full((_LANES,), -jnp.inf, jnp.float32)
                si = jnp.zeros((_LANES,), jnp.int32)
                for j in range(_N_GROUPS):
                    c2 = (bi != j) & (g[j] > sv)
                    sv = jnp.where(c2, g[j], sv)
                    si = jnp.where(c2, j, si)
                a_v, a_i, b_v, b_i = [], [], [], []
                sub_rows = sub_base + rows
                for o in range(_GROUP_SIZE):
                    ia = bi * _GROUP_SIZE + o
                    ib = si * _GROUP_SIZE + o
                    a_i.append(ia)
                    b_i.append(ib)
                    a_v.append(
                        plsc.load_gather(p_v, [ia * _SC_SUB + sub_rows]))
                    b_v.append(
                        plsc.load_gather(p_v, [ib * _SC_SUB + sub_rows]))
                for aa, bb in _SORT8:
                    _ce(a_v, a_i, aa, bb, lex=False)
                    _ce(b_v, b_i, aa, bb, lex=False)
                l_v, l_i = [], []
                for k in range(_TOP_K):
                    c3 = _lex_gt(a_v[k], a_i[k], b_v[7 - k], b_i[7 - k])
                    l_v.append(jnp.where(c3, a_v[k], b_v[7 - k]))
                    l_i.append(jnp.where(c3, a_i[k], b_i[7 - k]))
                for aa, bb in _BMERGE8:
                    _ce(l_v, l_i, aa, bb)
                pos_base = (q * 2 + c // 8) * (_TOP_K * 128) + (c % 8) * 16
                for k in range(_TOP_K):
                    wout_v[pl.ds(pos_base + k * 128, _LANES)] = l_v[k]
                    iout_v[pl.ds(pos_base + k * 128, _LANES)] = l_i[k]
                    plsc.addupdate_scatter(cnt_v, [l_i[k]], ones)
                return carry

            return chunk

        out_sz = _SC_SUB * _TOP_K
        for q in range(n_sub):
            in_copies[q].wait()
            lax.fori_loop(0, _SC_SUB // _LANES, make_chunk(q), 0)
            out_copies.append(pltpu.async_copy(
                wout_v.at[pl.ds(q * out_sz, out_sz)],
                w_hbm.at[pl.ds(base * _TOP_K + q * out_sz, out_sz)],
                so))
            out_copies.append(pltpu.async_copy(
                iout_v.at[pl.ds(q * out_sz, out_sz)],
                idx_hbm.at[pl.ds(base * _TOP_K + q * out_sz, out_sz)],
                so))
        pltpu.sync_copy(cnt_v, cnt_hbm.at[wid])
        for h in out_copies:
            h.wait()

    return body(p_flat)


def _finalize_body(wf_ref, if_ref, cnt_ref, w_ref, i_ref, c_ref, *, bf):
    w_ref[...] = jnp.concatenate(
        [wf_ref[j].T for j in range(bf)], axis=0)
    i_ref[...] = jnp.concatenate(
        [if_ref[j].T for j in range(bf)], axis=0)

    @pl.when(pl.program_id(0) == 0)
    def _():
        c_ref[...] = jnp.sum(cnt_ref[...], axis=0)


def _finalize(w_flat, idx_flat, cnt_parts, n, n_e, bf=4):
    nw = cnt_parts.shape[0]
    w3 = w_flat.reshape(n // 128, _TOP_K, 128)
    idx3 = idx_flat.reshape(n // 128, _TOP_K, 128)
    return pl.pallas_call(
        functools.partial(_finalize_body, bf=bf),
        grid=(n // (128 * bf),),
        in_specs=[
            pl.BlockSpec((bf, _TOP_K, 128), lambda i: (i, 0, 0)),
            pl.BlockSpec((bf, _TOP_K, 128), lambda i: (i, 0, 0)),
            pl.BlockSpec((nw, n_e), lambda i: (0, 0)),
        ],
        out_specs=[
            pl.BlockSpec((128 * bf, _TOP_K), lambda i: (i, 0)),
            pl.BlockSpec((128 * bf, _TOP_K), lambda i: (i, 0)),
            pl.BlockSpec((n_e,), lambda i: (0,)),
        ],
        out_shape=[
            jax.ShapeDtypeStruct((n, _TOP_K), jnp.float32),
            jax.ShapeDtypeStruct((n, _TOP_K), jnp.int32),
            jax.ShapeDtypeStruct((n_e,), jnp.float32),
        ],
    )(w3, idx3, cnt_parts)


def kernel(x, w_gate, halves=1):
    n = x.shape[0]
    n_e = w_gate.shape[0]
    nh = n // halves
    outs = []
    for h in range(halves):
        p = _gate_softmax(x, w_gate, h * nh, nh)
        outs.append(_route_sc(p, nh, n_e))
    w_flat = jnp.concatenate([o[0] for o in outs])
    idx_flat = jnp.concatenate([o[1] for o in outs])
    cnt_parts = jnp.concatenate([o[2] for o in outs], axis=0)
    weights, topk_indices, counts = _finalize(
        w_flat, idx_flat, cnt_parts, n, n_e)
    return (weights.astype(x.dtype), topk_indices, counts)


# speedup vs baseline: 1.0348x; 1.0348x over previous
import functools

import jax
import jax.numpy as jnp
from jax import lax
from jax.experimental import pallas as pl
from jax.experimental.pallas import tpu as pltpu
from jax.experimental.pallas import tpu_sc as plsc

_TOP_K = 8
_N_GROUPS = 8
_GROUP_SIZE = 8
_LANES = 16
_SC_SUB = 256

_SORT8 = (
    (0, 1), (2, 3), (4, 5), (6, 7),
    (0, 2), (1, 3), (4, 6), (5, 7),
    (1, 2), (5, 6), (0, 4), (3, 7),
    (1, 5), (2, 6),
    (1, 4), (3, 6),
    (2, 4), (3, 5),
    (3, 4),
)
_BMERGE8 = (
    (0, 4), (1, 5), (2, 6), (3, 7),
    (0, 2), (1, 3), (4, 6), (5, 7),
    (0, 1), (2, 3), (4, 5), (6, 7),
)


def _gate_softmax_body(x_ref, w_ref, p_ref, *, sc_tok):
    s = lax.dot_general(
        w_ref[...], x_ref[...],
        (((1,), (1,)), ((), ())),
        preferred_element_type=jnp.float32,
    )
    m = jnp.max(s, axis=0, keepdims=True)
    e = jnp.exp(s - m)
    p = e / jnp.sum(e, axis=0, keepdims=True)
    block = p.shape[1]
    p_ref[...] = jnp.concatenate(
        [p[:, j * sc_tok:(j + 1) * sc_tok].reshape(-1, 128)
         for j in range(block // sc_tok)], axis=0)


def _gate_softmax(x, w_gate, tok_base, n_tok, block=1024):
    n, d = x.shape
    n_e = w_gate.shape[0]
    rows_per_block = n_e * block // 128
    base_blocks = tok_base // block
    return pl.pallas_call(
        functools.partial(_gate_softmax_body, sc_tok=_SC_SUB),
        grid=(n_tok // block,),
        in_specs=[
            pl.BlockSpec((block, d), lambda i: (i + base_blocks, 0)),
            pl.BlockSpec((n_e, d), lambda i: (0, 0)),
        ],
        out_specs=pl.BlockSpec((rows_per_block, 128), lambda i: (i, 0)),
        out_shape=jax.ShapeDtypeStruct((n_e * n_tok // 128, 128),
                                       jnp.float32),
    )(x, w_gate)


def _lex_gt(av, ai, bv, bi):
    return (av > bv) | ((av == bv) & (ai < bi))


def _ce(vals, idxs, a, b, lex=True):
    if lex:
        c = _lex_gt(vals[a], idxs[a], vals[b], idxs[b])
    else:
        c = vals[a] > vals[b]
    va = jnp.where(c, vals[a], vals[b])
    vb = jnp.where(c, vals[b], vals[a])
    ia = jnp.where(c, idxs[a], idxs[b])
    ib = jnp.where(c, idxs[b], idxs[a])
    vals[a], vals[b], idxs[a], idxs[b] = va, vb, ia, ib


def _route_sc(p3, n, n_e):
    info = plsc.get_sparse_core_info()
    nw = info.num_cores * info.num_subcores
    tok_w = n // nw
    n_chunks = tok_w // _LANES
    mesh = plsc.VectorSubcoreMesh(core_axis_name="c", subcore_axis_name="s")
    p_flat = p3.reshape(n * n_e)

    @functools.partial(
        pl.kernel,
        mesh=mesh,
        compiler_params=pltpu.CompilerParams(needs_layout_passes=False),
        out_type=[
            jax.ShapeDtypeStruct((n * _TOP_K,), jnp.float32),
            jax.ShapeDtypeStruct((n * _TOP_K,), jnp.int32),
            jax.ShapeDtypeStruct((nw, n_e), jnp.float32),
        ],
        scratch_types=[
            pltpu.VMEM((n_e * tok_w,), jnp.float32),
            pltpu.VMEM((tok_w * _TOP_K,), jnp.float32),
            pltpu.VMEM((tok_w * _TOP_K,), jnp.int32),
            pltpu.VMEM((n_e,), jnp.float32),
            pltpu.SemaphoreType.DMA,
            pltpu.SemaphoreType.DMA,
            pltpu.SemaphoreType.DMA,
            pltpu.SemaphoreType.DMA,
            pltpu.SemaphoreType.DMA,
        ],
    )
    def body(pt_hbm, w_hbm, idx_hbm, cnt_hbm, p_v, wout_v, iout_v, cnt_v,
             s0, s1, s2, s3, so):
        n_sub = tok_w // _SC_SUB
        sub_sz = n_e * _SC_SUB
        in_sems = [s0, s1, s2, s3]
        cid = lax.axis_index("c")
        sid = lax.axis_index("s")
        wid = sid * info.num_cores + cid
        base = wid * tok_w
        in_copies = []
        for q in range(n_sub):
            in_copies.append(pltpu.async_copy(
                pt_hbm.at[pl.ds((wid * n_sub + q) * sub_sz, sub_sz)],
                p_v.at[pl.ds(q * sub_sz, sub_sz)],
                in_sems[q]))
        zeros = jnp.zeros((_LANES,), jnp.float32)
        for j in range(n_e // _LANES):
            cnt_v[pl.ds(j * _LANES, _LANES)] = zeros
        lanes = lax.iota(jnp.int32, _LANES)
        ones = jnp.ones((_LANES,), jnp.float32)
        out_copies = []

        def make_chunk(q):
            sub_base = q * sub_sz

            def chunk(c, carry):
                col = c * _LANES
                rows = col + lanes
                g = []
                for j in range(_N_GROUPS):
                    m = p_v[pl.ds(
                        sub_base + j * _GROUP_SIZE * _SC_SUB + col, _LANES)]
                    for o in range(1, _GROUP_SIZE):
                        m = jnp.maximum(
                            m,
                            p_v[pl.ds(
                                sub_base + (j * _GROUP_SIZE + o) * _SC_SUB
                                + col, _LANES)])
                    g.append(m)
                bv, bi = g[0], jnp.zeros((_LANES,), jnp.int32)
                for j in range(1, _N_GROUPS):
                    c1 = g[j] > bv
                    bv = jnp.where(c1, g[j], bv)
                    bi = jnp.where(c1, j, bi)
                sv = jnp.full((_LANES,), -jnp.inf, jnp.float32)
                si = jnp.zeros((_LANES,), jnp.int32)
                for j in range(_N_GROUPS):
                    c2 = (bi != j) & (g[j] > sv)
                    sv = jnp.where(c2, g[j], sv)
                    si = jnp.where(c2, j, si)
                a_v, a_i, b_v, b_i = [], [], [], []
                sub_rows = sub_base + rows
                for o in range(_GROUP_SIZE):
                    ia = bi * _GROUP_SIZE + o
                    ib = si * _GROUP_SIZE + o
                    a_i.append(ia)
                    b_i.append(ib)
                    a_v.append(
                        plsc.load_gather(p_v, [ia * _SC_SUB + sub_rows]))
                    b_v.append(
                        plsc.load_gather(p_v, [ib * _SC_SUB + sub_rows]))
                for aa, bb in _SORT8:
                    _ce(a_v, a_i, aa, bb, lex=False)
                    _ce(b_v, b_i, aa, bb, lex=False)
                l_v, l_i = [], []
                for k in range(_TOP_K):
                    c3 = _lex_gt(a_v[k], a_i[k], b_v[7 - k], b_i[7 - k])
                    l_v.append(jnp.where(c3, a_v[k], b_v[7 - k]))
                    l_i.append(jnp.where(c3, a_i[k], b_i[7 - k]))
                for aa, bb in _BMERGE8:
                    _ce(l_v, l_i, aa, bb)
                out_pos = (q * _SC_SUB + rows) * _TOP_K
                for k in range(_TOP_K):
                    plsc.store_scatter(wout_v, [out_pos + k], l_v[k])
                    plsc.store_scatter(iout_v, [out_pos + k], l_i[k])
                    plsc.addupdate_scatter(cnt_v, [l_i[k]], ones)
                return carry

            return chunk

        out_sz = _SC_SUB * _TOP_K
        for q in range(n_sub):
            in_copies[q].wait()
            lax.fori_loop(0, _SC_SUB // _LANES, make_chunk(q), 0)
            out_copies.append(pltpu.async_copy(
                wout_v.at[pl.ds(q * out_sz, out_sz)],
                w_hbm.at[pl.ds(base * _TOP_K + q * out_sz, out_sz)],
                so))
            out_copies.append(pltpu.async_copy(
                iout_v.at[pl.ds(q * out_sz, out_sz)],
                idx_hbm.at[pl.ds(base * _TOP_K + q * out_sz, out_sz)],
                so))
        pltpu.sync_copy(cnt_v, cnt_hbm.at[wid])
        for h in out_copies:
            h.wait()

    return body(p_flat)


def kernel(x, w_gate, halves=2):
    n = x.shape[0]
    n_e = w_gate.shape[0]
    nh = n // halves
    outs = []
    for h in range(halves):
        p = _gate_softmax(x, w_gate, h * nh, nh)
        outs.append(_route_sc(p, nh, n_e))
    weights = jnp.concatenate(
        [o[0].reshape(nh, _TOP_K) for o in outs]).astype(x.dtype)
    topk_indices = jnp.concatenate(
        [o[1].reshape(nh, _TOP_K) for o in outs])
    cnt_parts = jnp.concatenate([o[2] for o in outs], axis=0)
    counts = jnp.sum(cnt_parts, axis=0)
    return (weights, topk_indices, counts)


# speedup vs baseline: 1.0758x; 1.0396x over previous
import functools

import jax
import jax.numpy as jnp
from jax import lax
from jax.experimental import pallas as pl
from jax.experimental.pallas import tpu as pltpu
from jax.experimental.pallas import tpu_sc as plsc

_TOP_K = 8
_N_GROUPS = 8
_GROUP_SIZE = 8
_LANES = 16
_SC_SUB = 256

_SORT8 = (
    (0, 1), (2, 3), (4, 5), (6, 7),
    (0, 2), (1, 3), (4, 6), (5, 7),
    (1, 2), (5, 6), (0, 4), (3, 7),
    (1, 5), (2, 6),
    (1, 4), (3, 6),
    (2, 4), (3, 5),
    (3, 4),
)
_BMERGE8 = (
    (0, 4), (1, 5), (2, 6), (3, 7),
    (0, 2), (1, 3), (4, 6), (5, 7),
    (0, 1), (2, 3), (4, 5), (6, 7),
)


def _gate_softmax_body(x_ref, w_ref, p_ref, *, sc_tok):
    s = lax.dot_general(
        w_ref[...], x_ref[...],
        (((1,), (1,)), ((), ())),
        preferred_element_type=jnp.float32,
    )
    m = jnp.max(s, axis=0, keepdims=True)
    e = jnp.exp(s - m)
    p = e / jnp.sum(e, axis=0, keepdims=True)
    block = p.shape[1]
    p_ref[...] = jnp.concatenate(
        [p[:, j * sc_tok:(j + 1) * sc_tok].reshape(-1, 128)
         for j in range(block // sc_tok)], axis=0)


def _gate_softmax(x, w_gate, tok_base, n_tok, block=1024):
    n, d = x.shape
    n_e = w_gate.shape[0]
    rows_per_block = n_e * block // 128
    base_blocks = tok_base // block
    return pl.pallas_call(
        functools.partial(_gate_softmax_body, sc_tok=_SC_SUB),
        grid=(n_tok // block,),
        in_specs=[
            pl.BlockSpec((block, d), lambda i: (i + base_blocks, 0)),
            pl.BlockSpec((n_e, d), lambda i: (0, 0)),
        ],
        out_specs=pl.BlockSpec((rows_per_block, 128), lambda i: (i, 0)),
        out_shape=jax.ShapeDtypeStruct((n_e * n_tok // 128, 128),
                                       jnp.float32),
    )(x, w_gate)


def _lex_gt(av, ai, bv, bi):
    return (av > bv) | ((av == bv) & (ai < bi))


def _ce(vals, idxs, a, b, lex=True):
    if lex:
        c = _lex_gt(vals[a], idxs[a], vals[b], idxs[b])
    else:
        c = vals[a] > vals[b]
    va = jnp.where(c, vals[a], vals[b])
    vb = jnp.where(c, vals[b], vals[a])
    ia = jnp.where(c, idxs[a], idxs[b])
    ib = jnp.where(c, idxs[b], idxs[a])
    vals[a], vals[b], idxs[a], idxs[b] = va, vb, ia, ib


def _route_sc(p3, n, n_e):
    info = plsc.get_sparse_core_info()
    nw = info.num_cores * info.num_subcores
    tok_w = n // nw
    n_chunks = tok_w // _LANES
    mesh = plsc.VectorSubcoreMesh(core_axis_name="c", subcore_axis_name="s")
    p_flat = p3.reshape(n * n_e)

    @functools.partial(
        pl.kernel,
        mesh=mesh,
        compiler_params=pltpu.CompilerParams(needs_layout_passes=False),
        out_type=[
            jax.ShapeDtypeStruct((n * _TOP_K,), jnp.float32),
            jax.ShapeDtypeStruct((n * _TOP_K,), jnp.int32),
            jax.ShapeDtypeStruct((nw, n_e), jnp.float32),
        ],
        scratch_types=[
            pltpu.VMEM((n_e * tok_w,), jnp.float32),
            pltpu.VMEM((tok_w * _TOP_K,), jnp.float32),
            pltpu.VMEM((tok_w * _TOP_K,), jnp.int32),
            pltpu.VMEM((n_e,), jnp.float32),
            pltpu.SemaphoreType.DMA,
            pltpu.SemaphoreType.DMA,
            pltpu.SemaphoreType.DMA,
            pltpu.SemaphoreType.DMA,
            pltpu.SemaphoreType.DMA,
        ],
    )
    def body(pt_hbm, w_hbm, idx_hbm, cnt_hbm, p_v, wout_v, iout_v, cnt_v,
             s0, s1, s2, s3, so):
        n_sub = tok_w // _SC_SUB
        sub_sz = n_e * _SC_SUB
        in_sems = [s0, s1, s2, s3]
        cid = lax.axis_index("c")
        sid = lax.axis_index("s")
        wid = sid * info.num_cores + cid
        base = wid * tok_w
        in_copies = []
        for q in range(n_sub):
            in_copies.append(pltpu.async_copy(
                pt_hbm.at[pl.ds((wid * n_sub + q) * sub_sz, sub_sz)],
                p_v.at[pl.ds(q * sub_sz, sub_sz)],
                in_sems[q]))
        zeros = jnp.zeros((_LANES,), jnp.float32)
        for j in range(n_e // _LANES):
            cnt_v[pl.ds(j * _LANES, _LANES)] = zeros
        lanes = lax.iota(jnp.int32, _LANES)
        ones = jnp.ones((_LANES,), jnp.float32)
        out_copies = []

        def make_chunk(q):
            sub_base = q * sub_sz

            def chunk(c, carry):
                col = c * _LANES
                rows = col + lanes
                g = []
                for j in range(_N_GROUPS):
                    m = p_v[pl.ds(
                        sub_base + j * _GROUP_SIZE * _SC_SUB + col, _LANES)]
                    for o in range(1, _GROUP_SIZE):
                        m = jnp.maximum(
                            m,
                            p_v[pl.ds(
                                sub_base + (j * _GROUP_SIZE + o) * _SC_SUB
                                + col, _LANES)])
                    g.append(m)
                bv, bi = g[0], jnp.zeros((_LANES,), jnp.int32)
                for j in range(1, _N_GROUPS):
                    c1 = g[j] > bv
                    bv = jnp.where(c1, g[j], bv)
                    bi = jnp.where(c1, j, bi)
                sv = jnp.full((_LANES,), -jnp.inf, jnp.float32)
                si = jnp.zeros((_LANES,), jnp.int32)
                for j in range(_N_GROUPS):
                    c2 = (bi != j) & (g[j] > sv)
                    sv = jnp.where(c2, g[j], sv)
                    si = jnp.where(c2, j, si)
                a_v, a_i, b_v, b_i = [], [], [], []
                sub_rows = sub_base + rows
                for o in range(_GROUP_SIZE):
                    ia = bi * _GROUP_SIZE + o
                    ib = si * _GROUP_SIZE + o
                    a_i.append(ia)
                    b_i.append(ib)
                    a_v.append(
                        plsc.load_gather(p_v, [ia * _SC_SUB + sub_rows]))
                    b_v.append(
                        plsc.load_gather(p_v, [ib * _SC_SUB + sub_rows]))
                for aa, bb in _SORT8:
                    _ce(a_v, a_i, aa, bb, lex=False)
                    _ce(b_v, b_i, aa, bb, lex=False)
                l_v, l_i = [], []
                for k in range(_TOP_K):
                    c3 = _lex_gt(a_v[k], a_i[k], b_v[7 - k], b_i[7 - k])
                    l_v.append(jnp.where(c3, a_v[k], b_v[7 - k]))
                    l_i.append(jnp.where(c3, a_i[k], b_i[7 - k]))
                for aa, bb in _BMERGE8:
                    _ce(l_v, l_i, aa, bb)
                out_pos = (q * _SC_SUB + rows) * _TOP_K
                for k in range(_TOP_K):
                    plsc.store_scatter(wout_v, [out_pos + k], l_v[k])
                    plsc.store_scatter(iout_v, [out_pos + k], l_i[k])
                    plsc.addupdate_scatter(cnt_v, [l_i[k]], ones)
                return carry

            return chunk

        out_sz = _SC_SUB * _TOP_K
        for q in range(n_sub):
            in_copies[q].wait()
            lax.fori_loop(0, _SC_SUB // _LANES, make_chunk(q), 0)
            out_copies.append(pltpu.async_copy(
                wout_v.at[pl.ds(q * out_sz, out_sz)],
                w_hbm.at[pl.ds(base * _TOP_K + q * out_sz, out_sz)],
                so))
            out_copies.append(pltpu.async_copy(
                iout_v.at[pl.ds(q * out_sz, out_sz)],
                idx_hbm.at[pl.ds(base * _TOP_K + q * out_sz, out_sz)],
                so))
        pltpu.sync_copy(cnt_v, cnt_hbm.at[wid])
        for h in out_copies:
            h.wait()

    return body(p_flat)


def kernel(x, w_gate, halves=1):
    n = x.shape[0]
    n_e = w_gate.shape[0]
    nh = n // halves
    outs = []
    for h in range(halves):
        p = _gate_softmax(x, w_gate, h * nh, nh)
        outs.append(_route_sc(p, nh, n_e))
    w_flat = jnp.concatenate([o[0] for o in outs])
    idx_flat = jnp.concatenate([o[1] for o in outs])
    cnt_parts = jnp.concatenate([o[2] for o in outs], axis=0)
    weights = w_flat.reshape(n, _TOP_K).astype(x.dtype)
    topk_indices = idx_flat.reshape(n, _TOP_K)
    counts = jnp.sum(cnt_parts, axis=0)
    return (weights, topk_indices, counts)
